# baseline (speedup 1.0000x reference)
import functools

import jax
import jax.numpy as jnp
from jax import lax
from jax.experimental import pallas as pl
from jax.experimental.pallas import tpu as pltpu
from jax.experimental.pallas import tpu_sc as plsc

B, S, D = 4, 4096, 2048
ROWS = B * S

NC, NS, L = 2, 16, 16
NW = NC * NS
RPT = ROWS // NW
G = 16
NG = RPT // G


def _body(emb_hbm, mask_hbm, mte_hbm, out_hbm,
          mask_v, uidx_v, midx_v, mte_rep, buf0, buf1,
          sem_g, sem_s, sem_m):
    wid = lax.axis_index("s") * NC + lax.axis_index("c")
    base = wid * RPT

    pltpu.sync_copy(mask_hbm.at[pl.ds(base, RPT)], mask_v)
    for r in range(G):
        pltpu.sync_copy(mte_hbm, mte_rep.at[r])

    iota16 = lax.iota(jnp.int32, L)
    zeros16 = jnp.zeros((L,), jnp.int32)

    def scan16(v):
        for k in (1, 2, 4, 8):
            idx = jnp.maximum(iota16 - k, 0)
            g = lax.gather(
                v, idx[:, None],
                lax.GatherDimensionNumbers(
                    offset_dims=(), collapsed_slice_dims=(0,),
                    start_index_map=(0,)),
                slice_sizes=(1,),
                mode=lax.GatherScatterMode.PROMISE_IN_BOUNDS)
            v = v + jnp.where(iota16 >= k, g, 0)
        return v

    def comp_body(g, carry):
        ucnt, mcnt = carry
        off = pl.multiple_of(g * G, G)
        m = mask_v[pl.ds(off, G)]
        unm = m == 0
        ids = base + off + iota16
        unm_i = unm.astype(jnp.int32)
        ucs = scan16(unm_i)
        mcs = (iota16 + 1) - ucs
        upos = jnp.where(unm, ucnt + ucs - 1, RPT)
        mpos = jnp.where(unm, RPT, mcnt + mcs - 1)
        plsc.store_scatter(uidx_v, [upos], ids)
        plsc.store_scatter(midx_v, [mpos], ids)
        pu = ucs[L - 1]
        return ucnt + pu, mcnt + (G - pu)

    ucnt, mcnt = lax.fori_loop(
        0, NG, comp_body, (jnp.int32(0), jnp.int32(0)))

    bufs = (buf0, buf1)

    for b in range(NG):
        @pl.when(b * G < ucnt)
        def _u(b=b):
            v = uidx_v[pl.ds(b * G, G)]
            vpad = plsc.load_gather(uidx_v, [zeros16])
            vi = jnp.where((b * G + iota16) < ucnt, v, vpad)
            buf = bufs[b % 2]
            pltpu.async_copy(emb_hbm.at[vi], buf, sem_g).wait()
            pltpu.async_copy(buf, out_hbm.at[vi], sem_s).wait()

    for b in range(NG):
        @pl.when(b * G < mcnt)
        def _m(b=b):
            v = midx_v[pl.ds(b * G, G)]
            vpad = plsc.load_gather(midx_v, [zeros16])
            vi = jnp.where((b * G + iota16) < mcnt, v, vpad)
            pltpu.async_copy(mte_rep, out_hbm.at[vi], sem_m).wait()


_sc_call = functools.partial(
    pl.kernel,
    out_type=jax.ShapeDtypeStruct((ROWS, D), jnp.float32),
    mesh=plsc.VectorSubcoreMesh(
        core_axis_name="c", subcore_axis_name="s",
        num_cores=NC, num_subcores=NS),
    compiler_params=pltpu.CompilerParams(needs_layout_passes=False),
    scratch_types=[
        pltpu.VMEM((RPT,), jnp.int32),
        pltpu.VMEM((RPT + G,), jnp.int32),
        pltpu.VMEM((RPT + G,), jnp.int32),
        pltpu.VMEM((G, D), jnp.float32),
        pltpu.VMEM((G, D), jnp.float32),
        pltpu.VMEM((G, D), jnp.float32),
        pltpu.SemaphoreType.DMA,
        pltpu.SemaphoreType.DMA,
        pltpu.SemaphoreType.DMA,
    ],
)(_body)


def kernel(embeddings, token_mask, mask_token_embed):
    emb = embeddings.reshape(ROWS, D)
    mask = token_mask.reshape(ROWS).astype(jnp.int32)
    out = _sc_call(emb, mask, mask_token_embed)
    return out.reshape(B, S, D)


# speedup vs baseline: 1.1164x; 1.1164x over previous
import functools

import jax
import jax.numpy as jnp
from jax import lax
from jax.experimental import pallas as pl
from jax.experimental.pallas import tpu as pltpu
from jax.experimental.pallas import tpu_sc as plsc

B, S, D = 4, 4096, 2048
ROWS = B * S

NC, NS, L = 2, 16, 16
NW = NC * NS
RPT = ROWS // NW
G = 16
NG = RPT // G
NBUF = 2
MNB = 8


def _body(emb_hbm, mask_hbm, mte_hbm, out_hbm,
          mask_v, uidx_v, midx_v, mte_rep, gbuf,
          sem_g, sem_s, sem_m):
    wid = lax.axis_index("s") * NC + lax.axis_index("c")
    base = wid * RPT

    pltpu.sync_copy(mask_hbm.at[pl.ds(base, RPT)], mask_v)
    for r in range(G):
        pltpu.sync_copy(mte_hbm, mte_rep.at[r])

    iota16 = lax.iota(jnp.int32, L)
    zeros16 = jnp.zeros((L,), jnp.int32)

    def scan16(v):
        for k in (1, 2, 4, 8):
            idx = jnp.maximum(iota16 - k, 0)
            g = lax.gather(
                v, idx[:, None],
                lax.GatherDimensionNumbers(
                    offset_dims=(), collapsed_slice_dims=(0,),
                    start_index_map=(0,)),
                slice_sizes=(1,),
                mode=lax.GatherScatterMode.PROMISE_IN_BOUNDS)
            v = v + jnp.where(iota16 >= k, g, 0)
        return v

    def comp_body(g, carry):
        ucnt, mcnt = carry
        off = pl.multiple_of(g * G, G)
        m = mask_v[pl.ds(off, G)]
        unm = m == 0
        ids = base + off + iota16
        unm_i = unm.astype(jnp.int32)
        ucs = scan16(unm_i)
        mcs = (iota16 + 1) - ucs
        upos = jnp.where(unm, ucnt + ucs - 1, RPT)
        mpos = jnp.where(unm, RPT, mcnt + mcs - 1)
        plsc.store_scatter(uidx_v, [upos], ids)
        plsc.store_scatter(midx_v, [mpos], ids)
        pu = ucs[L - 1]
        return ucnt + pu, mcnt + (G - pu)

    ucnt, mcnt = lax.fori_loop(
        0, NG, comp_body, (jnp.int32(0), jnp.int32(0)))

    nb_u = (ucnt + G - 1) // G
    nb_m = (mcnt + G - 1) // G
    nb_max = jnp.maximum(nb_u, nb_m)

    def batch_vi(idx_ref, cnt, b):
        v = idx_ref[pl.ds(b * G, G)]
        vpad = plsc.load_gather(idx_ref, [zeros16])
        return jnp.where((b * G + iota16) < cnt, v, vpad)

    def slot_ref(b):
        off = pl.multiple_of(lax.rem(b, NBUF) * G, G)
        return gbuf.at[pl.ds(off, G)]

    def wait_g():
        pltpu.make_async_copy(emb_hbm.at[zeros16], slot_ref(0), sem_g).wait()

    def wait_s():
        pltpu.make_async_copy(slot_ref(0), out_hbm.at[zeros16], sem_s).wait()

    def wait_m():
        pltpu.make_async_copy(mte_rep, out_hbm.at[zeros16], sem_m).wait()

    def loop_body(b, c):
        @pl.when(b < nb_m)
        def _fm():
            vim = batch_vi(midx_v, mcnt, b)
            pltpu.async_copy(mte_rep, out_hbm.at[vim], sem_m)

        @pl.when(jnp.logical_and(b >= MNB, b - MNB < nb_m))
        def _wm():
            wait_m()

        @pl.when(b < nb_u)
        def _u():
            vi = batch_vi(uidx_v, ucnt, b)
            pltpu.async_copy(emb_hbm.at[vi], slot_ref(b), sem_g)
            wait_g()

            @pl.when(b >= 1)
            def _ws():
                wait_s()

            pltpu.async_copy(slot_ref(b), out_hbm.at[vi], sem_s)

        return c

    lax.fori_loop(0, nb_max, loop_body, 0)

    @pl.when(nb_u > 0)
    def _ep():
        wait_s()

    waited = jnp.clip(nb_max - MNB, 0, nb_m)

    def drain_m(i, c):
        wait_m()
        return c

    lax.fori_loop(0, nb_m - waited, drain_m, 0)


_sc_call = functools.partial(
    pl.kernel,
    out_type=jax.ShapeDtypeStruct((ROWS, D), jnp.float32),
    mesh=plsc.VectorSubcoreMesh(
        core_axis_name="c", subcore_axis_name="s",
        num_cores=NC, num_subcores=NS),
    compiler_params=pltpu.CompilerParams(needs_layout_passes=False),
    scratch_types=[
        pltpu.VMEM((RPT,), jnp.int32),
        pltpu.VMEM((RPT + G,), jnp.int32),
        pltpu.VMEM((RPT + G,), jnp.int32),
        pltpu.VMEM((G, D), jnp.float32),
        pltpu.VMEM((NBUF * G, D), jnp.float32),
        pltpu.SemaphoreType.DMA,
        pltpu.SemaphoreType.DMA,
        pltpu.SemaphoreType.DMA,
    ],
)(_body)


def kernel(embeddings, token_mask, mask_token_embed):
    emb = embeddings.reshape(ROWS, D)
    mask = token_mask.reshape(ROWS).astype(jnp.int32)
    out = _sc_call(emb, mask, mask_token_embed)
    return out.reshape(B, S, D)
